# Initial kernel scaffold; baseline (speedup 1.0000x reference)
import jax
import jax.numpy as jnp
from jax.experimental import pallas as pl


def kernel(Xi, Xv, tables, Wb0, bb0, Wb1, bb1, Wb2, bb2, Wt0, bt0, Wt1, bt1, Wt2, bt2, Wt3, bt3):
    raise NotImplementedError("write your pallas kernel here")


# speedup vs baseline: 2.2179x; 2.2179x over previous
import functools

import jax
import jax.numpy as jnp
import numpy as np
from jax import lax
from jax.experimental import pallas as pl
from jax.experimental.pallas import tpu as pltpu
from jax.experimental.pallas import tpu_sc as plsc

B = 4096
FIELDS = 26
VOCAB = 100000
EMB = 32
NUM = 13
NF1 = FIELDS + 1

NC = 2
NS = 16
NW = NC * NS
ROWS = B * FIELDS
RPW = ROWS // NW
CH = 128
NCH = RPW // CH
FIRE = 13

_sc_mesh = plsc.VectorSubcoreMesh(core_axis_name="c", subcore_axis_name="s")


@functools.partial(
    pl.kernel,
    out_type=jax.ShapeDtypeStruct((ROWS, EMB), jnp.float32),
    mesh=_sc_mesh,
    scratch_types=[
        pltpu.VMEM((NCH, CH), jnp.int32),
        pltpu.VMEM((RPW, EMB), jnp.float32),
        pltpu.SemaphoreType.DMA,
    ],
    compiler_params=pltpu.CompilerParams(use_tc_tiling_on_sc=False),
)
def _sc_gather(table_hbm, idx_hbm, out_hbm, idx_v, rows_v, sem):
    wid = lax.axis_index("s") * NC + lax.axis_index("c")
    pltpu.sync_copy(idx_hbm.at[wid], idx_v)
    for g in range(NCH // FIRE):
        handles = []
        for k in range(g * FIRE, (g + 1) * FIRE):
            handles.append(
                pltpu.async_copy(
                    table_hbm.at[idx_v.at[k]],
                    rows_v.at[pl.ds(k * CH, CH)],
                    sem,
                )
            )
        for h in handles:
            h.wait()
    pltpu.sync_copy(rows_v, out_hbm.at[pl.ds(wid * RPW, RPW)])


BS = 512
GRID = B // BS


def _dense_body(Xv_ref, E_ref, Wb0_ref, bb0_ref, Wb1_ref, bb1_ref,
                Wb2_ref, bb2_ref, Wt0x_ref, Wt0z_ref, bt0_ref,
                Wt1_ref, bt1_ref, Wt2_ref, bt2_ref, Wt3_ref, bt3_ref,
                o_ref):
    f32 = jnp.float32
    x = Xv_ref[...]
    x = jnp.maximum(jnp.dot(x, Wb0_ref[...], preferred_element_type=f32)
                    + bb0_ref[...], 0.0)
    x = jnp.maximum(jnp.dot(x, Wb1_ref[...], preferred_element_type=f32)
                    + bb1_ref[...], 0.0)
    x = jnp.maximum(jnp.dot(x, Wb2_ref[...], preferred_element_type=f32)
                    + bb2_ref[...], 0.0)
    T3 = jnp.concatenate([x[:, None, :], E_ref[...]], axis=1)
    Z = lax.dot_general(T3, T3, (((2,), (2,)), ((0,), (0,))),
                        preferred_element_type=f32)
    Zr = Z.reshape(BS, NF1 * NF1)
    h = jnp.dot(x, Wt0x_ref[...], preferred_element_type=f32)
    h = h + jnp.dot(Zr, Wt0z_ref[...], preferred_element_type=f32)
    h = jnp.maximum(h + bt0_ref[...], 0.0)
    h = jnp.maximum(jnp.dot(h, Wt1_ref[...], preferred_element_type=f32)
                    + bt1_ref[...], 0.0)
    h = jnp.maximum(jnp.dot(h, Wt2_ref[...], preferred_element_type=f32)
                    + bt2_ref[...], 0.0)
    o = jnp.dot(h, Wt3_ref[...], preferred_element_type=f32) + bt3_ref[...]
    o_ref[...] = 1.0 / (1.0 + jnp.exp(-o))


def _full(shape):
    return pl.BlockSpec(shape, lambda i: tuple(0 for _ in shape))


_dense_call = pl.pallas_call(
    _dense_body,
    grid=(GRID,),
    in_specs=[
        pl.BlockSpec((BS, NUM), lambda i: (i, 0)),
        pl.BlockSpec((BS, FIELDS, EMB), lambda i: (i, 0, 0)),
        _full((NUM, 512)), _full((1, 512)),
        _full((512, 256)), _full((1, 256)),
        _full((256, EMB)), _full((1, EMB)),
        _full((EMB, 1024)), _full((NF1 * NF1, 1024)), _full((1, 1024)),
        _full((1024, 512)), _full((1, 512)),
        _full((512, 256)), _full((1, 256)),
        _full((256, 1)), _full((1, 1)),
    ],
    out_specs=pl.BlockSpec((BS, 1), lambda i: (i, 0)),
    out_shape=jax.ShapeDtypeStruct((B, 1), jnp.float32),
)


@jax.jit
def kernel(Xi, Xv, tables, Wb0, bb0, Wb1, bb1, Wb2, bb2,
           Wt0, bt0, Wt1, bt1, Wt2, bt2, Wt3, bt3):
    flat_tables = tables.reshape(FIELDS * VOCAB, EMB)
    flat_idx = (Xi + jnp.arange(FIELDS, dtype=jnp.int32)[None, :] * VOCAB)
    idx3 = flat_idx.reshape(NW, NCH, CH)
    embs = _sc_gather(flat_tables, idx3)
    E3 = embs.reshape(B, FIELDS, EMB)

    li, lj = np.tril_indices(NF1, k=-1)
    rows = jnp.asarray(li * NF1 + lj, dtype=jnp.int32)
    Wt0z = jnp.zeros((NF1 * NF1, 1024), jnp.float32).at[rows].set(Wt0[EMB:])

    out = _dense_call(
        Xv, E3,
        Wb0, bb0.reshape(1, -1), Wb1, bb1.reshape(1, -1),
        Wb2, bb2.reshape(1, -1),
        Wt0[:EMB], Wt0z, bt0.reshape(1, -1),
        Wt1, bt1.reshape(1, -1), Wt2, bt2.reshape(1, -1),
        Wt3, bt3.reshape(1, -1),
    )
    return out


# speedup vs baseline: 3.4652x; 1.5624x over previous
import functools

import jax
import jax.numpy as jnp
import numpy as np
from jax import lax
from jax.experimental import pallas as pl
from jax.experimental.pallas import tpu as pltpu
from jax.experimental.pallas import tpu_sc as plsc

B = 4096
FIELDS = 26
VOCAB = 100000
EMB = 32
NUM = 13
NF1 = FIELDS + 1

VB = 8192
NVB = -(-VOCAB // VB)


def _repack_body(tt_ref, out_ref):
    blk = tt_ref[0]
    t = jnp.transpose(blk)
    out_ref[0] = jnp.pad(t, ((0, 0), (0, 128 - EMB)))


_repack_call = pl.pallas_call(
    _repack_body,
    grid=(FIELDS, NVB),
    in_specs=[pl.BlockSpec((1, EMB, VB), lambda f, v: (f, 0, v))],
    out_specs=pl.BlockSpec((1, VB, 128), lambda f, v: (f, v, 0)),
    out_shape=jax.ShapeDtypeStruct((FIELDS, VOCAB, 128), jnp.float32),
)

NC = 2
NS = 16
NW = NC * NS
ROWS = B * FIELDS
RPW = ROWS // NW
CH = 128
NCH = RPW // CH
NBUF = 2

_sc_mesh = plsc.VectorSubcoreMesh(core_axis_name="c", subcore_axis_name="s")


@functools.partial(
    pl.kernel,
    out_type=jax.ShapeDtypeStruct((ROWS, 128), jnp.float32),
    mesh=_sc_mesh,
    scratch_types=[
        pltpu.VMEM((NCH, CH), jnp.int32),
        pltpu.VMEM((NBUF, CH, 128), jnp.float32),
        pltpu.SemaphoreType.DMA,
        pltpu.SemaphoreType.DMA,
    ],
)
def _sc_gather(table_hbm, idx_hbm, out_hbm, idx_v, wide_v, gsem, osem):
    wid = lax.axis_index("s") * NC + lax.axis_index("c")
    base = wid * RPW
    pltpu.sync_copy(idx_hbm.at[wid], idx_v)
    gathers = []
    outs = []

    def drain_and_store(k):
        gathers[k].wait()
        outs.append(
            pltpu.async_copy(
                wide_v.at[k % NBUF],
                out_hbm.at[pl.ds(base + k * CH, CH)],
                osem,
            )
        )

    for k in range(NCH):
        b = k % NBUF
        if k >= NBUF:
            outs[k - NBUF].wait()
        gathers.append(
            pltpu.async_copy(table_hbm.at[idx_v.at[k]], wide_v.at[b], gsem)
        )
        if k >= NBUF - 1:
            drain_and_store(k - NBUF + 1)
    for k in range(NCH - NBUF + 1, NCH):
        drain_and_store(k)
    for h in outs[NCH - NBUF:]:
        h.wait()


BS = 512
GRID = B // BS


def _dense_body(Xv_ref, E_ref, Wb0_ref, bb0_ref, Wb1_ref, bb1_ref,
                Wb2_ref, bb2_ref, Wt0x_ref, Wt0z_ref, bt0_ref,
                Wt1_ref, bt1_ref, Wt2_ref, bt2_ref, Wt3_ref, bt3_ref,
                o_ref):
    f32 = jnp.float32
    x = Xv_ref[...]
    x = jnp.maximum(jnp.dot(x, Wb0_ref[...], preferred_element_type=f32)
                    + bb0_ref[...], 0.0)
    x = jnp.maximum(jnp.dot(x, Wb1_ref[...], preferred_element_type=f32)
                    + bb1_ref[...], 0.0)
    x = jnp.maximum(jnp.dot(x, Wb2_ref[...], preferred_element_type=f32)
                    + bb2_ref[...], 0.0)
    E = E_ref[..., 0:EMB]
    T3 = jnp.concatenate([x[:, None, :], E], axis=1)
    Z = lax.dot_general(T3, T3, (((2,), (2,)), ((0,), (0,))),
                        preferred_element_type=f32)
    Zr = Z.reshape(BS, NF1 * NF1)
    h = jnp.dot(x, Wt0x_ref[...], preferred_element_type=f32)
    h = h + jnp.dot(Zr, Wt0z_ref[...], preferred_element_type=f32)
    h = jnp.maximum(h + bt0_ref[...], 0.0)
    h = jnp.maximum(jnp.dot(h, Wt1_ref[...], preferred_element_type=f32)
                    + bt1_ref[...], 0.0)
    h = jnp.maximum(jnp.dot(h, Wt2_ref[...], preferred_element_type=f32)
                    + bt2_ref[...], 0.0)
    o = jnp.dot(h, Wt3_ref[...], preferred_element_type=f32) + bt3_ref[...]
    o_ref[...] = 1.0 / (1.0 + jnp.exp(-o))


def _full(shape):
    return pl.BlockSpec(shape, lambda i: tuple(0 for _ in shape))


_dense_call = pl.pallas_call(
    _dense_body,
    grid=(GRID,),
    in_specs=[
        pl.BlockSpec((BS, NUM), lambda i: (i, 0)),
        pl.BlockSpec((BS, FIELDS, 128), lambda i: (i, 0, 0)),
        _full((NUM, 512)), _full((1, 512)),
        _full((512, 256)), _full((1, 256)),
        _full((256, EMB)), _full((1, EMB)),
        _full((EMB, 1024)), _full((NF1 * NF1, 1024)), _full((1, 1024)),
        _full((1024, 512)), _full((1, 512)),
        _full((512, 256)), _full((1, 256)),
        _full((256, 1)), _full((1, 1)),
    ],
    out_specs=pl.BlockSpec((BS, 1), lambda i: (i, 0)),
    out_shape=jax.ShapeDtypeStruct((B, 1), jnp.float32),
)


@jax.jit
def kernel(Xi, Xv, tables, Wb0, bb0, Wb1, bb1, Wb2, bb2,
           Wt0, bt0, Wt1, bt1, Wt2, bt2, Wt3, bt3):
    tt = tables.transpose(0, 2, 1)
    packed = _repack_call(tt)
    flat_packed = packed.reshape(FIELDS * VOCAB, 128)

    flat_idx = Xi + jnp.arange(FIELDS, dtype=jnp.int32)[None, :] * VOCAB
    idx3 = flat_idx.reshape(NW, NCH, CH)
    embs = _sc_gather(flat_packed, idx3)
    E3 = embs.reshape(B, FIELDS, 128)

    li, lj = np.tril_indices(NF1, k=-1)
    rows = jnp.asarray(li * NF1 + lj, dtype=jnp.int32)
    Wt0z = jnp.zeros((NF1 * NF1, 1024), jnp.float32).at[rows].set(Wt0[EMB:])

    out = _dense_call(
        Xv, E3,
        Wb0, bb0.reshape(1, -1), Wb1, bb1.reshape(1, -1),
        Wb2, bb2.reshape(1, -1),
        Wt0[:EMB], Wt0z, bt0.reshape(1, -1),
        Wt1, bt1.reshape(1, -1), Wt2, bt2.reshape(1, -1),
        Wt3, bt3.reshape(1, -1),
    )
    return out
